# Initial kernel scaffold; baseline (speedup 1.0000x reference)
import jax
import jax.numpy as jnp
from jax.experimental import pallas as pl


def kernel(points, params):
    raise NotImplementedError("write your pallas kernel here")


# speedup vs baseline: 2.8505x; 2.8505x over previous
import functools

import jax
import jax.numpy as jnp
from jax.experimental import pallas as pl
from jax.experimental.pallas import tpu as pltpu

_F32 = jnp.float32


def _dotT(a, b, precision=None):
    return jax.lax.dot_general(a, b, (((1,), (1,)), ((), ())),
                               preferred_element_type=_F32,
                               precision=precision)


_EXACT = jax.lax.Precision.HIGHEST


def _onehot_pick(sel, row):
    vals = jnp.sum(sel * jnp.broadcast_to(row, sel.shape), axis=1,
                   keepdims=True)
    return jnp.transpose(vals)


def _bto(x, shape):
    return jnp.broadcast_to(x, shape)


def _cumsum_lanes(x):
    n = x.shape[-1]
    k = 1
    while k < n:
        pad = jnp.zeros(x.shape[:-1] + (k,), x.dtype)
        x = x + jnp.concatenate([pad, x[..., : n - k]], axis=-1)
        k *= 2
    return x


def _mlp(x, layers):
    for (W, g, b) in layers:
        x = jnp.dot(W, x, preferred_element_type=_F32)
        x = jax.nn.relu(g.reshape(-1, 1) * x + b.reshape(-1, 1))
    return x


def _fps(xs, ys, zs, npoint, F_ref):
    n = xs.shape[1]
    iota = jax.lax.broadcasted_iota(jnp.int32, (1, n), 1)

    def body(i, carry):
        dists, far = carry
        oh = iota == far
        F_ref[pl.ds(i, 1), :] = oh.astype(_F32)
        px = jnp.sum(jnp.where(oh, xs, 0.0))
        py = jnp.sum(jnp.where(oh, ys, 0.0))
        pz = jnp.sum(jnp.where(oh, zs, 0.0))
        d = (xs - px) ** 2 + (ys - py) ** 2 + (zs - pz) ** 2
        dists = jnp.minimum(dists, d)
        m = jnp.max(dists)
        far2 = jnp.min(jnp.where(dists == m, iota, n)).astype(jnp.int32)
        return dists, far2

    dists0 = jnp.full((1, n), 1e10, _F32)
    jax.lax.fori_loop(0, npoint, body, (dists0, jnp.int32(0)))
    return F_ref[...]


def _sa_layer(xs, ys, zs, feats, F_ref, M_ref, npoint, r2, nsample, layers):
    n = xs.shape[1]
    F = _fps(xs, ys, zs, npoint, F_ref)
    nxr = _onehot_pick(F, xs)
    nyr = _onehot_pick(F, ys)
    nzr = _onehot_pick(F, zs)
    nxc = jnp.transpose(nxr)
    nyc = jnp.transpose(nyr)
    nzc = jnp.transpose(nzr)
    sh = (npoint, n)
    d2 = ((_bto(nxc, sh) - _bto(xs, sh)) ** 2
          + (_bto(nyc, sh) - _bto(ys, sh)) ** 2
          + (_bto(nzc, sh) - _bto(zs, sh)) ** 2)
    mask = (d2 <= r2).astype(_F32)
    cum = _cumsum_lanes(mask)
    cnt = jnp.sum(mask, axis=1, keepdims=True)
    ncr = jnp.concatenate([nxr, nyr, nzr], axis=0)
    cf = jnp.concatenate([jnp.concatenate([xs, ys, zs], axis=0), feats],
                         axis=0)

    def gbody(s, carry):
        sp1 = (s + 1).astype(_F32)
        target = jnp.where(cnt >= sp1, sp1, 1.0)
        sel = ((mask > 0.0) & (cum == _bto(target, sh))).astype(_F32)
        g = _dotT(cf, sel, _EXACT)
        g = jnp.concatenate([g[0:3] - ncr, g[3:]], axis=0)
        h = _mlp(g, layers)

        @pl.when(s == 0)
        def _():
            M_ref[...] = h

        @pl.when(s != 0)
        def _():
            M_ref[...] = jnp.maximum(M_ref[...], h)

        return carry

    jax.lax.fori_loop(0, nsample, gbody, 0)
    return nxr, nyr, nzr, M_ref[...]


def _fp_layer(c1, c2, fea1, fea2, layers):
    x1, y1, z1 = c1
    x2, y2, z2 = c2
    n1 = x1.shape[1]
    n2 = x2.shape[1]
    sh = (n1, n2)
    x1c = jnp.transpose(x1)
    y1c = jnp.transpose(y1)
    z1c = jnp.transpose(z1)
    d2 = ((_bto(x1c, sh) - _bto(x2, sh)) ** 2
          + (_bto(y1c, sh) - _bto(y2, sh)) ** 2
          + (_bto(z1c, sh) - _bto(z2, sh)) ** 2)
    iota2 = jax.lax.broadcasted_iota(jnp.int32, sh, 1)
    dw = d2
    ohs = []
    drs = []
    for _ in range(3):
        m = jnp.min(dw, axis=1, keepdims=True)
        jm = jnp.min(jnp.where(dw == _bto(m, sh), iota2, n2),
                     axis=1, keepdims=True)
        oh = (iota2 == _bto(jm, sh)).astype(_F32)
        ohs.append(oh)
        drs.append(1.0 / (m + 1e-8))
        dw = jnp.where(oh > 0.0, 1e30, dw)
    denom = drs[0] + drs[1] + drs[2]
    interp = None
    for k in range(3):
        w = drs[k] / denom
        part = _dotT(fea2, ohs[k] * _bto(w, sh), _EXACT)
        interp = part if interp is None else interp + part
    nf = interp if fea1 is None else jnp.concatenate([interp, fea1], axis=0)
    return _mlp(nf, layers)


def _forward_body(treedef, nleaves, *args):
    pts_ref = args[0]
    wrefs = args[1:1 + nleaves]
    o0, o1, o2 = args[1 + nleaves:4 + nleaves]
    F1, F2, F3, F4, M1, M2, M3, M4 = args[4 + nleaves:]

    vals = [r[...] for r in wrefs]
    p = jax.tree_util.tree_unflatten(treedef, vals)

    xyzT = jnp.transpose(pts_ref[0])
    xs = xyzT[0:1]
    ys = xyzT[1:2]
    zs = xyzT[2:3]

    x1, y1, z1, f1 = _sa_layer(xs, ys, zs, xyzT, F1, M1, 256, 0.01, 16, p['sa1'])
    x2, y2, z2, f2 = _sa_layer(x1, y1, z1, f1, F2, M2, 128, 0.01, 16, p['sa2'])
    x3, y3, z3, f3 = _sa_layer(x2, y2, z2, f2, F3, M3, 64, 0.04, 16, p['sa3'])
    x4, y4, z4, f4 = _sa_layer(x3, y3, z3, f3, F4, M4, 16, 0.16, 16, p['sa4'])

    f3 = _fp_layer((x3, y3, z3), (x4, y4, z4), f3, f4, p['fp4'])
    f2 = _fp_layer((x2, y2, z2), (x3, y3, z3), f2, f3, p['fp3'])
    f1 = _fp_layer((x1, y1, z1), (x2, y2, z2), f1, f2, p['fp2'])
    l0 = _fp_layer((xs, ys, zs), (x1, y1, z1), None, f1, p['fp1'])

    W, g, b = p['shared']
    h = jax.nn.relu(g.reshape(-1, 1) * jnp.dot(W, l0, preferred_element_type=_F32)
                    + b.reshape(-1, 1))
    Wo, bo = p['off']
    off = jnp.dot(Wo, h, preferred_element_type=_F32) + bo.reshape(-1, 1)
    Wc, bc = p['cls']
    score = jax.nn.sigmoid(jnp.dot(Wc, h, preferred_element_type=_F32)
                           + bc.reshape(-1, 1))

    o0[0] = jnp.transpose(l0)
    o1[0] = score
    o2[0] = jnp.transpose(off)


def kernel(points, params):
    B, N, _ = points.shape
    leaves, treedef = jax.tree_util.tree_flatten(params)
    ins = [l.reshape(1, -1) if l.ndim == 1 else l for l in leaves]
    treedef2 = jax.tree_util.tree_structure(params)

    def const_spec(a):
        nd = a.ndim
        return pl.BlockSpec(a.shape, lambda b, nd=nd: (0,) * nd)

    body = functools.partial(_forward_body, treedef2, len(ins))
    out0, out1, out2 = pl.pallas_call(
        body,
        grid=(B,),
        in_specs=[pl.BlockSpec((1, N, 3), lambda b: (b, 0, 0))]
                 + [const_spec(a) for a in ins],
        out_specs=[
            pl.BlockSpec((1, N, 128), lambda b: (b, 0, 0)),
            pl.BlockSpec((1, 1, N), lambda b: (b, 0, 0)),
            pl.BlockSpec((1, N, 3), lambda b: (b, 0, 0)),
        ],
        out_shape=[
            jax.ShapeDtypeStruct((B, N, 128), _F32),
            jax.ShapeDtypeStruct((B, 1, N), _F32),
            jax.ShapeDtypeStruct((B, N, 3), _F32),
        ],
        scratch_shapes=[
            pltpu.VMEM((256, N), _F32),
            pltpu.VMEM((128, 256), _F32),
            pltpu.VMEM((64, 128), _F32),
            pltpu.VMEM((16, 64), _F32),
            pltpu.VMEM((64, 256), _F32),
            pltpu.VMEM((128, 128), _F32),
            pltpu.VMEM((256, 64), _F32),
            pltpu.VMEM((512, 16), _F32),
        ],
    )(points, *ins)
    return out0, out1.reshape(B, N), out2
